# baseline (speedup 1.0000x reference)
import functools

import jax
import jax.numpy as jnp
from jax import lax
from jax.experimental import pallas as pl
from jax.experimental.pallas import tpu as pltpu
from jax.experimental.pallas import tpu_sc as plsc

N = 10000
NP = 10240
E = 320000
D = 128
H = 256
C = 10
G = 64

F32 = jnp.float32


def _edge_body(a_ref, w_ref, b_ref, o_ref):
    o_ref[...] = (
        jnp.dot(a_ref[...], w_ref[0], preferred_element_type=F32) + b_ref[0]
    )


@functools.cache
def _edge_kernel(h, te=2000):
    ne = E // te
    return pl.pallas_call(
        _edge_body,
        grid=(2, ne),
        in_specs=[
            pl.BlockSpec((te, D), lambda c, i: (i, 0)),
            pl.BlockSpec((1, D, h), lambda c, i: (c, 0, 0)),
            pl.BlockSpec((1, 1, h), lambda c, i: (c, 0, 0)),
        ],
        out_specs=pl.BlockSpec((te, h), lambda c, i: (c * ne + i, 0)),
        out_shape=jax.ShapeDtypeStruct((2 * E, h), F32),
    )


def _edge_body_full(a_ref, w_ref, b_ref, o_ref):
    o_ref[...] = (
        jnp.dot(a_ref[...], w_ref[...], preferred_element_type=F32) + b_ref[...]
    )


@functools.cache
def _edge_kernel_full(te=2000):
    ne = E // te
    return pl.pallas_call(
        _edge_body_full,
        grid=(ne,),
        in_specs=[
            pl.BlockSpec((te, D), lambda i: (i, 0)),
            pl.BlockSpec((D, D), lambda i: (0, 0)),
            pl.BlockSpec((1, D), lambda i: (0, 0)),
        ],
        out_specs=pl.BlockSpec((te, D), lambda i: (i, 0)),
        out_shape=jax.ShapeDtypeStruct((E, D), F32),
    )


@functools.cache
def _sc_aggr_kernel(h, edge_split):
    k = 80
    nrows = E // k
    nchs = 128 if edge_split else 256
    rp = NP // 16
    zr = 8
    mesh = plsc.VectorSubcoreMesh(core_axis_name="c", subcore_axis_name="s")

    @functools.partial(
        pl.kernel,
        out_type=jax.ShapeDtypeStruct((2 * NP, h), F32),
        mesh=mesh,
        scratch_types=[
            pltpu.VMEM((4, k), jnp.int32),
            pltpu.VMEM((4, k), jnp.int32),
            pltpu.VMEM((k, h), F32),
            pltpu.VMEM((k, h), F32),
            pltpu.VMEM((k, h), F32),
            pltpu.VMEM((k, h), F32),
            pltpu.VMEM((zr, h), F32),
            pltpu.VMEM_SHARED((NP, h), F32),
            pltpu.SemaphoreType.DMA,
            pltpu.SemaphoreType.DMA,
            pltpu.SemaphoreType.DMA,
            pltpu.SemaphoreType.DMA,
            pltpu.SemaphoreType.DMA,
            pltpu.SemaphoreType.DMA,
            pltpu.SemaphoreType.DMA,
            pltpu.SemaphoreType.DMA,
        ],
    )
    def sc_aggr(x_hbm, e_hbm, src_hbm, dst_hbm, out_hbm,
                srcv, dstv, xg0, xg1, eb0, eb1, zb, aggr,
                sem_is, sem_id, se0, se1, sg0, sg1, ss0, ss1):
        c = lax.axis_index("c")
        s = lax.axis_index("s")
        if edge_split:
            w = c * 16 + s
            rowbase = pl.multiple_of(w * nchs, nchs)
            ecoff = 0
            coff = 0
        else:
            rowbase = pl.multiple_of(s * nchs, nchs)
            ecoff = c * E
            coff = c * NP
        nch = jnp.minimum(nrows - rowbase, nchs)

        xgs = (xg0, xg1)
        ebs = (eb0, eb1)
        ses = (se0, se1)
        sgs = (sg0, sg1)
        sss = (ss0, ss1)

        def idx_issue(j):
            off = pl.multiple_of((rowbase + j) * k, k)
            q = j & 3
            pltpu.async_copy(src_hbm.at[pl.ds(off, k)], srcv.at[q], sem_is)
            pltpu.async_copy(dst_hbm.at[pl.ds(off, k)], dstv.at[q], sem_id)

        def idx_wait():
            pltpu.make_async_copy(src_hbm.at[pl.ds(0, k)], srcv.at[0], sem_is).wait()
            pltpu.make_async_copy(dst_hbm.at[pl.ds(0, k)], dstv.at[0], sem_id).wait()

        def adj(q):
            if not edge_split:
                for jj in range(k // 16):
                    srcv[q, pl.ds(jj * 16, 16)] = (
                        srcv[q, pl.ds(jj * 16, 16)] + coff
                    )

        def gather_issue(j, b):
            pltpu.async_copy(x_hbm.at[srcv.at[j & 3]], xgs[b], sgs[b])

        def e_issue(j, b):
            off = pl.multiple_of(ecoff + (rowbase + j) * k, k)
            pltpu.async_copy(e_hbm.at[pl.ds(off, k)], ebs[b], ses[b])

        idx_issue(0)
        idx_wait()
        adj(0)
        gather_issue(0, 0)
        e_issue(0, 0)
        idx_issue(1)

        for r in range(zr):
            for jj in range(h // 16):
                zb[r, pl.ds(jj * 16, 16)] = jnp.zeros((16,), F32)

        def zloop(kk, carry):
            pltpu.sync_copy(zb, aggr.at[pl.ds(s * rp + kk * zr, zr)])
            return carry

        lax.fori_loop(0, rp // zr, zloop, 0)
        plsc.subcore_barrier()

        def outer(j0, carry):
            for b in range(2):
                j = j0 * 2 + b
                b1 = 1 - b

                @pl.when(j < nch)
                def _():
                    @pl.when(j + 1 < nch)
                    def _():
                        q1 = (j + 1) & 3
                        idx_wait()
                        adj(q1)
                        gather_issue(j + 1, b1)

                        @pl.when(j + 2 < nch)
                        def _():
                            idx_issue(j + 2)

                        @pl.when(j >= 1)
                        def _():
                            pltpu.make_async_copy(
                                ebs[b1], aggr.at[dstv.at[0]], sss[b1]
                            ).wait()

                        e_issue(j + 1, b1)

                    pltpu.make_async_copy(
                        e_hbm.at[pl.ds(0, k)], ebs[b], ses[b]
                    ).wait()
                    pltpu.make_async_copy(
                        x_hbm.at[srcv.at[0]], xgs[b], sgs[b]
                    ).wait()

                    def rloop(r, rc):
                        for jj in range(h // 16):
                            v = (
                                xgs[b][r, pl.ds(jj * 16, 16)]
                                + ebs[b][r, pl.ds(jj * 16, 16)]
                            )
                            ebs[b][r, pl.ds(jj * 16, 16)] = jnp.maximum(v, 0.0)
                        return rc

                    lax.fori_loop(0, k, rloop, 0)
                    pltpu.async_copy(
                        ebs[b], aggr.at[dstv.at[j & 3]], sss[b], add=True
                    )
            return carry

        lax.fori_loop(0, (nch + 1) // 2, outer, 0)
        pltpu.make_async_copy(ebs[0], aggr.at[dstv.at[0]], sss[0]).wait()
        pltpu.make_async_copy(ebs[1], aggr.at[dstv.at[0]], sss[1]).wait()
        plsc.subcore_barrier()
        pltpu.sync_copy(
            aggr.at[pl.ds(s * rp, rp)], out_hbm.at[pl.ds(c * NP + s * rp, rp)]
        )

    return sc_aggr


def _mlp_body(xl, xh, al, ah, w1a, w1b, b1, w2, b2, z_ref, st_ref):
    i = pl.program_id(0)
    t = xl.shape[0]
    hin_lo = xl[...] + al[...]
    hin_hi = xh[...] + ah[...]
    h1 = jnp.maximum(
        jnp.dot(hin_lo, w1a[...], preferred_element_type=F32)
        + jnp.dot(hin_hi, w1b[...], preferred_element_type=F32)
        + b1[...],
        0.0,
    )
    z = jnp.dot(h1, w2[...], preferred_element_type=F32) + b2[...]
    rows = lax.broadcasted_iota(jnp.int32, (t, 1), 0) + i * t
    z = jnp.where(rows < N, z, 0.0)
    z_ref[...] = z
    st = jnp.concatenate(
        [jnp.sum(z, 0, keepdims=True), jnp.sum(z * z, 0, keepdims=True)], 0
    )

    @pl.when(i == 0)
    def _():
        st_ref[...] = st

    @pl.when(i > 0)
    def _():
        st_ref[...] += st


@functools.cache
def _mlp_kernel(h, t=2048):
    nt = NP // t
    return pl.pallas_call(
        _mlp_body,
        grid=(nt,),
        in_specs=[
            pl.BlockSpec((t, h), lambda i: (i, 0)),
            pl.BlockSpec((t, h), lambda i: (NP // t + i, 0)),
            pl.BlockSpec((t, h), lambda i: (i, 0)),
            pl.BlockSpec((t, h), lambda i: (NP // t + i, 0)),
            pl.BlockSpec((h, H), lambda i: (0, 0)),
            pl.BlockSpec((h, H), lambda i: (0, 0)),
            pl.BlockSpec((1, H), lambda i: (0, 0)),
            pl.BlockSpec((H, H), lambda i: (0, 0)),
            pl.BlockSpec((1, H), lambda i: (0, 0)),
        ],
        out_specs=[
            pl.BlockSpec((t, H), lambda i: (i, 0)),
            pl.BlockSpec((2, H), lambda i: (0, 0)),
        ],
        out_shape=[
            jax.ShapeDtypeStruct((NP, H), F32),
            jax.ShapeDtypeStruct((2, H), F32),
        ],
    )


def _mlp_body0(xr, a0, a1, w1, b1, w2, b2, z_ref, st_ref):
    i = pl.program_id(0)
    t = xr.shape[0]
    hin = xr[...] + a0[...] + a1[...]
    h1 = jnp.maximum(
        jnp.dot(hin, w1[...], preferred_element_type=F32) + b1[...], 0.0
    )
    z = jnp.dot(h1, w2[...], preferred_element_type=F32) + b2[...]
    rows = lax.broadcasted_iota(jnp.int32, (t, 1), 0) + i * t
    z = jnp.where(rows < N, z, 0.0)
    z_ref[...] = z
    st = jnp.concatenate(
        [jnp.sum(z, 0, keepdims=True), jnp.sum(z * z, 0, keepdims=True)], 0
    )

    @pl.when(i == 0)
    def _():
        st_ref[...] = st

    @pl.when(i > 0)
    def _():
        st_ref[...] += st


@functools.cache
def _mlp_kernel0(t=2048):
    nt = NP // t
    return pl.pallas_call(
        _mlp_body0,
        grid=(nt,),
        in_specs=[
            pl.BlockSpec((t, D), lambda i: (i, 0)),
            pl.BlockSpec((t, D), lambda i: (i, 0)),
            pl.BlockSpec((t, D), lambda i: (NP // t + i, 0)),
            pl.BlockSpec((D, H), lambda i: (0, 0)),
            pl.BlockSpec((1, H), lambda i: (0, 0)),
            pl.BlockSpec((H, H), lambda i: (0, 0)),
            pl.BlockSpec((1, H), lambda i: (0, 0)),
        ],
        out_specs=[
            pl.BlockSpec((t, H), lambda i: (i, 0)),
            pl.BlockSpec((2, H), lambda i: (0, 0)),
        ],
        out_shape=[
            jax.ShapeDtypeStruct((NP, H), F32),
            jax.ShapeDtypeStruct((2, H), F32),
        ],
    )


def _norm_body(z_ref, st_ref, g_ref, bt_ref, o_ref):
    i = pl.program_id(1)
    t = z_ref.shape[0]
    mu = st_ref[0:1, :] * (1.0 / N)
    var = st_ref[1:2, :] * (1.0 / N) - mu * mu
    inv = lax.rsqrt(var + 1e-5)
    o = jnp.maximum((z_ref[...] - mu) * inv * g_ref[...] + bt_ref[...], 0.0)
    rows = lax.broadcasted_iota(jnp.int32, (t, 1), 0) + i * t
    o_ref[...] = jnp.where(rows < N, o, 0.0)


@functools.cache
def _norm_kernel(t=2048):
    nt = NP // t
    hh = H // 2
    return pl.pallas_call(
        _norm_body,
        grid=(2, nt),
        in_specs=[
            pl.BlockSpec((t, hh), lambda c, i: (i, c)),
            pl.BlockSpec((2, hh), lambda c, i: (0, c)),
            pl.BlockSpec((1, hh), lambda c, i: (0, c)),
            pl.BlockSpec((1, hh), lambda c, i: (0, c)),
        ],
        out_specs=pl.BlockSpec((t, hh), lambda c, i: (c * (NP // t) + i, 0)),
        out_shape=jax.ShapeDtypeStruct((2 * NP, hh), F32),
    )


def _pool_body(xl, xh, b_ref, wl, bl, o_ref, acc, cnt):
    i = pl.program_id(0)
    nt = pl.num_programs(0)
    t = b_ref.shape[0]
    hh = H // 2
    oh = (b_ref[...] == lax.broadcasted_iota(jnp.int32, (t, G), 1)).astype(F32)

    @pl.when(i == 0)
    def _():
        acc[...] = jnp.zeros((G, H), F32)
        cnt[...] = jnp.zeros((G, 1), F32)

    dn = (((0,), (0,)), ((), ()))
    acc[:, 0:hh] += lax.dot_general(oh, xl[...], dn, preferred_element_type=F32)
    acc[:, hh:H] += lax.dot_general(oh, xh[...], dn, preferred_element_type=F32)
    cnt[...] += lax.dot_general(
        oh, jnp.ones((t, 1), F32), dn, preferred_element_type=F32
    )

    @pl.when(i == nt - 1)
    def _():
        pooled = acc[...] / jnp.maximum(cnt[...], 1.0)
        o_ref[...] = (
            jnp.dot(pooled, wl[...], preferred_element_type=F32) + bl[...]
        )


@functools.cache
def _pool_kernel(t=2048):
    nt = NP // t
    hh = H // 2
    return pl.pallas_call(
        _pool_body,
        grid=(nt,),
        in_specs=[
            pl.BlockSpec((t, hh), lambda i: (i, 0)),
            pl.BlockSpec((t, hh), lambda i: (NP // t + i, 0)),
            pl.BlockSpec((t, 1), lambda i: (i, 0)),
            pl.BlockSpec((H, C), lambda i: (0, 0)),
            pl.BlockSpec((1, C), lambda i: (0, 0)),
        ],
        out_specs=pl.BlockSpec((G, C), lambda i: (0, 0)),
        out_shape=jax.ShapeDtypeStruct((G, C), F32),
        scratch_shapes=[
            pltpu.VMEM((G, H), F32),
            pltpu.VMEM((G, 1), F32),
        ],
    )


def kernel(x, edge_index, edge_attr, batch,
           We0, be0, W10, b10, W20, b20, g0, bt0,
           We1, be1, W11, b11, W21, b21, g1, bt1,
           We2, be2, W12, b12, W22, b22, g2, bt2,
           Wl, bl):
    src = edge_index[0]
    dst = edge_index[1]
    batch2 = jnp.concatenate(
        [batch, jnp.full((NP - N,), G, jnp.int32)]
    ).reshape(NP, 1)

    e0 = _edge_kernel_full()(edge_attr, We0, be0.reshape(1, D))
    hh = H // 2
    e_splits = []
    for We, be in ((We1, be1), (We2, be2)):
        We_s = jnp.stack([We[:, :hh], We[:, hh:]])
        be_s = jnp.stack([be[:hh], be[hh:]]).reshape(2, 1, hh)
        e_splits.append(_edge_kernel(hh)(edge_attr, We_s, be_s))

    xp = jnp.concatenate([x, jnp.zeros((NP - N, D), F32)], axis=0)
    aggr2 = _sc_aggr_kernel(D, True)(xp, e0, src, dst)
    z, stats = _mlp_kernel0()(
        xp, aggr2, aggr2, W10, b10.reshape(1, H), W20, b20.reshape(1, H)
    )
    x_split = _norm_kernel()(z, stats, g0.reshape(1, H), bt0.reshape(1, H))

    for li, (W1, b1, W2, b2, g, bt) in enumerate((
        (W11, b11, W21, b21, g1, bt1),
        (W12, b12, W22, b22, g2, bt2),
    )):
        aggr_split = _sc_aggr_kernel(hh, False)(x_split, e_splits[li], src, dst)
        z, stats = _mlp_kernel(hh)(
            x_split, x_split, aggr_split, aggr_split,
            W1[:hh], W1[hh:], b1.reshape(1, H), W2, b2.reshape(1, H),
        )
        x_split = _norm_kernel()(z, stats, g.reshape(1, H), bt.reshape(1, H))

    return _pool_kernel()(x_split, x_split, batch2, Wl, bl.reshape(1, C))


# speedup vs baseline: 1.0072x; 1.0072x over previous
import functools

import jax
import jax.numpy as jnp
from jax import lax
from jax.experimental import pallas as pl
from jax.experimental.pallas import tpu as pltpu
from jax.experimental.pallas import tpu_sc as plsc

N = 10000
NP = 10240
E = 320000
D = 128
H = 256
C = 10
G = 64

F32 = jnp.float32


def _edge_body(a_ref, w_ref, b_ref, o_ref):
    o_ref[...] = (
        jnp.dot(a_ref[...], w_ref[0], preferred_element_type=F32) + b_ref[0]
    )


@functools.cache
def _edge_kernel(h, te=2000):
    ne = E // te
    return pl.pallas_call(
        _edge_body,
        grid=(2, ne),
        in_specs=[
            pl.BlockSpec((te, D), lambda c, i: (i, 0)),
            pl.BlockSpec((1, D, h), lambda c, i: (c, 0, 0)),
            pl.BlockSpec((1, 1, h), lambda c, i: (c, 0, 0)),
        ],
        out_specs=pl.BlockSpec((te, h), lambda c, i: (c * ne + i, 0)),
        out_shape=jax.ShapeDtypeStruct((2 * E, h), F32),
    )


def _edge_body_full(a_ref, w_ref, b_ref, o_ref):
    o_ref[...] = (
        jnp.dot(a_ref[...], w_ref[...], preferred_element_type=F32) + b_ref[...]
    )


@functools.cache
def _edge_kernel_full(te=2000):
    ne = E // te
    return pl.pallas_call(
        _edge_body_full,
        grid=(ne,),
        in_specs=[
            pl.BlockSpec((te, D), lambda i: (i, 0)),
            pl.BlockSpec((D, D), lambda i: (0, 0)),
            pl.BlockSpec((1, D), lambda i: (0, 0)),
        ],
        out_specs=pl.BlockSpec((te, D), lambda i: (i, 0)),
        out_shape=jax.ShapeDtypeStruct((E, D), F32),
    )


@functools.cache
def _sc_aggr_kernel(h, edge_split):
    k = 80
    nrows = E // k
    nstr = 32 if edge_split else 16
    nch = nrows // nstr
    rp = NP // 16
    zr = 8
    mesh = plsc.VectorSubcoreMesh(core_axis_name="c", subcore_axis_name="s")

    @functools.partial(
        pl.kernel,
        out_type=jax.ShapeDtypeStruct((2 * NP, h), F32),
        mesh=mesh,
        scratch_types=[
            pltpu.VMEM((4, k), jnp.int32),
            pltpu.VMEM((4, k), jnp.int32),
            pltpu.VMEM((k, h), F32),
            pltpu.VMEM((k, h), F32),
            pltpu.VMEM((k, h), F32),
            pltpu.VMEM((k, h), F32),
            pltpu.VMEM((zr, h), F32),
            pltpu.VMEM_SHARED((NP, h), F32),
            pltpu.SemaphoreType.DMA,
            pltpu.SemaphoreType.DMA,
            pltpu.SemaphoreType.DMA,
            pltpu.SemaphoreType.DMA,
            pltpu.SemaphoreType.DMA,
            pltpu.SemaphoreType.DMA,
            pltpu.SemaphoreType.DMA,
            pltpu.SemaphoreType.DMA,
        ],
    )
    def sc_aggr(x_hbm, e_hbm, src_hbm, dst_hbm, out_hbm,
                srcv, dstv, xg0, xg1, eb0, eb1, zb, aggr,
                sem_is, sem_id, se0, se1, sg0, sg1, ss0, ss1):
        c = lax.axis_index("c")
        s = lax.axis_index("s")
        if edge_split:
            rowbase = c * 16 + s
            ecoff = 0
            coff = 0
        else:
            rowbase = s
            ecoff = c * E
            coff = c * NP

        xgs = (xg0, xg1)
        ebs = (eb0, eb1)
        ses = (se0, se1)
        sgs = (sg0, sg1)
        sss = (ss0, ss1)

        def idx_issue(j):
            off = pl.multiple_of((rowbase + j * nstr) * k, k)
            q = j & 3
            pltpu.async_copy(src_hbm.at[pl.ds(off, k)], srcv.at[q], sem_is)
            pltpu.async_copy(dst_hbm.at[pl.ds(off, k)], dstv.at[q], sem_id)

        def idx_wait():
            pltpu.make_async_copy(src_hbm.at[pl.ds(0, k)], srcv.at[0], sem_is).wait()
            pltpu.make_async_copy(dst_hbm.at[pl.ds(0, k)], dstv.at[0], sem_id).wait()

        def adj(q):
            if not edge_split:
                for jj in range(k // 16):
                    srcv[q, pl.ds(jj * 16, 16)] = (
                        srcv[q, pl.ds(jj * 16, 16)] + coff
                    )

        def gather_issue(j, b):
            pltpu.async_copy(x_hbm.at[srcv.at[j & 3]], xgs[b], sgs[b])

        def e_issue(j, b):
            off = pl.multiple_of(ecoff + (rowbase + j * nstr) * k, k)
            pltpu.async_copy(e_hbm.at[pl.ds(off, k)], ebs[b], ses[b])

        idx_issue(0)
        idx_wait()
        adj(0)
        gather_issue(0, 0)
        e_issue(0, 0)
        idx_issue(1)

        for r in range(zr):
            for jj in range(h // 16):
                zb[r, pl.ds(jj * 16, 16)] = jnp.zeros((16,), F32)

        def zloop(kk, carry):
            pltpu.sync_copy(zb, aggr.at[pl.ds(s * rp + kk * zr, zr)])
            return carry

        lax.fori_loop(0, rp // zr, zloop, 0)
        plsc.subcore_barrier()

        def outer(j0, carry):
            for b in range(2):
                j = j0 * 2 + b
                b1 = 1 - b

                @pl.when(j < nch)
                def _():
                    @pl.when(j + 1 < nch)
                    def _():
                        q1 = (j + 1) & 3
                        idx_wait()
                        adj(q1)
                        gather_issue(j + 1, b1)

                        @pl.when(j + 2 < nch)
                        def _():
                            idx_issue(j + 2)

                        @pl.when(j >= 1)
                        def _():
                            pltpu.make_async_copy(
                                ebs[b1], aggr.at[dstv.at[0]], sss[b1]
                            ).wait()

                        e_issue(j + 1, b1)

                    pltpu.make_async_copy(
                        e_hbm.at[pl.ds(0, k)], ebs[b], ses[b]
                    ).wait()
                    pltpu.make_async_copy(
                        x_hbm.at[srcv.at[0]], xgs[b], sgs[b]
                    ).wait()

                    def rloop(r4, rc):
                        for m in range(4):
                            r = r4 * 4 + m
                            for jj in range(h // 16):
                                v = (
                                    xgs[b][r, pl.ds(jj * 16, 16)]
                                    + ebs[b][r, pl.ds(jj * 16, 16)]
                                )
                                ebs[b][r, pl.ds(jj * 16, 16)] = jnp.maximum(
                                    v, 0.0
                                )
                        return rc

                    lax.fori_loop(0, k // 4, rloop, 0)
                    pltpu.async_copy(
                        ebs[b], aggr.at[dstv.at[j & 3]], sss[b], add=True
                    )
            return carry

        lax.fori_loop(0, (nch + 1) // 2, outer, 0)
        pltpu.make_async_copy(ebs[0], aggr.at[dstv.at[0]], sss[0]).wait()
        pltpu.make_async_copy(ebs[1], aggr.at[dstv.at[0]], sss[1]).wait()
        plsc.subcore_barrier()
        pltpu.sync_copy(
            aggr.at[pl.ds(s * rp, rp)], out_hbm.at[pl.ds(c * NP + s * rp, rp)]
        )

    return sc_aggr


def _mlp_body(xl, xh, al, ah, w1a, w1b, b1, w2, b2, z_ref, st_ref):
    i = pl.program_id(0)
    t = xl.shape[0]
    hin_lo = xl[...] + al[...]
    hin_hi = xh[...] + ah[...]
    h1 = jnp.maximum(
        jnp.dot(hin_lo, w1a[...], preferred_element_type=F32)
        + jnp.dot(hin_hi, w1b[...], preferred_element_type=F32)
        + b1[...],
        0.0,
    )
    z = jnp.dot(h1, w2[...], preferred_element_type=F32) + b2[...]
    rows = lax.broadcasted_iota(jnp.int32, (t, 1), 0) + i * t
    z = jnp.where(rows < N, z, 0.0)
    z_ref[...] = z
    st = jnp.concatenate(
        [jnp.sum(z, 0, keepdims=True), jnp.sum(z * z, 0, keepdims=True)], 0
    )

    @pl.when(i == 0)
    def _():
        st_ref[...] = st

    @pl.when(i > 0)
    def _():
        st_ref[...] += st


@functools.cache
def _mlp_kernel(h, t=2048):
    nt = NP // t
    return pl.pallas_call(
        _mlp_body,
        grid=(nt,),
        in_specs=[
            pl.BlockSpec((t, h), lambda i: (i, 0)),
            pl.BlockSpec((t, h), lambda i: (NP // t + i, 0)),
            pl.BlockSpec((t, h), lambda i: (i, 0)),
            pl.BlockSpec((t, h), lambda i: (NP // t + i, 0)),
            pl.BlockSpec((h, H), lambda i: (0, 0)),
            pl.BlockSpec((h, H), lambda i: (0, 0)),
            pl.BlockSpec((1, H), lambda i: (0, 0)),
            pl.BlockSpec((H, H), lambda i: (0, 0)),
            pl.BlockSpec((1, H), lambda i: (0, 0)),
        ],
        out_specs=[
            pl.BlockSpec((t, H), lambda i: (i, 0)),
            pl.BlockSpec((2, H), lambda i: (0, 0)),
        ],
        out_shape=[
            jax.ShapeDtypeStruct((NP, H), F32),
            jax.ShapeDtypeStruct((2, H), F32),
        ],
    )


def _mlp_body0(xr, a0, a1, w1, b1, w2, b2, z_ref, st_ref):
    i = pl.program_id(0)
    t = xr.shape[0]
    hin = xr[...] + a0[...] + a1[...]
    h1 = jnp.maximum(
        jnp.dot(hin, w1[...], preferred_element_type=F32) + b1[...], 0.0
    )
    z = jnp.dot(h1, w2[...], preferred_element_type=F32) + b2[...]
    rows = lax.broadcasted_iota(jnp.int32, (t, 1), 0) + i * t
    z = jnp.where(rows < N, z, 0.0)
    z_ref[...] = z
    st = jnp.concatenate(
        [jnp.sum(z, 0, keepdims=True), jnp.sum(z * z, 0, keepdims=True)], 0
    )

    @pl.when(i == 0)
    def _():
        st_ref[...] = st

    @pl.when(i > 0)
    def _():
        st_ref[...] += st


@functools.cache
def _mlp_kernel0(t=2048):
    nt = NP // t
    return pl.pallas_call(
        _mlp_body0,
        grid=(nt,),
        in_specs=[
            pl.BlockSpec((t, D), lambda i: (i, 0)),
            pl.BlockSpec((t, D), lambda i: (i, 0)),
            pl.BlockSpec((t, D), lambda i: (NP // t + i, 0)),
            pl.BlockSpec((D, H), lambda i: (0, 0)),
            pl.BlockSpec((1, H), lambda i: (0, 0)),
            pl.BlockSpec((H, H), lambda i: (0, 0)),
            pl.BlockSpec((1, H), lambda i: (0, 0)),
        ],
        out_specs=[
            pl.BlockSpec((t, H), lambda i: (i, 0)),
            pl.BlockSpec((2, H), lambda i: (0, 0)),
        ],
        out_shape=[
            jax.ShapeDtypeStruct((NP, H), F32),
            jax.ShapeDtypeStruct((2, H), F32),
        ],
    )


def _norm_body(z_ref, st_ref, g_ref, bt_ref, o_ref):
    i = pl.program_id(1)
    t = z_ref.shape[0]
    mu = st_ref[0:1, :] * (1.0 / N)
    var = st_ref[1:2, :] * (1.0 / N) - mu * mu
    inv = lax.rsqrt(var + 1e-5)
    o = jnp.maximum((z_ref[...] - mu) * inv * g_ref[...] + bt_ref[...], 0.0)
    rows = lax.broadcasted_iota(jnp.int32, (t, 1), 0) + i * t
    o_ref[...] = jnp.where(rows < N, o, 0.0)


@functools.cache
def _norm_kernel(t=2048):
    nt = NP // t
    hh = H // 2
    return pl.pallas_call(
        _norm_body,
        grid=(2, nt),
        in_specs=[
            pl.BlockSpec((t, hh), lambda c, i: (i, c)),
            pl.BlockSpec((2, hh), lambda c, i: (0, c)),
            pl.BlockSpec((1, hh), lambda c, i: (0, c)),
            pl.BlockSpec((1, hh), lambda c, i: (0, c)),
        ],
        out_specs=pl.BlockSpec((t, hh), lambda c, i: (c * (NP // t) + i, 0)),
        out_shape=jax.ShapeDtypeStruct((2 * NP, hh), F32),
    )


def _pool_body(xl, xh, b_ref, wl, bl, o_ref, acc, cnt):
    i = pl.program_id(0)
    nt = pl.num_programs(0)
    t = b_ref.shape[0]
    hh = H // 2
    oh = (b_ref[...] == lax.broadcasted_iota(jnp.int32, (t, G), 1)).astype(F32)

    @pl.when(i == 0)
    def _():
        acc[...] = jnp.zeros((G, H), F32)
        cnt[...] = jnp.zeros((G, 1), F32)

    dn = (((0,), (0,)), ((), ()))
    acc[:, 0:hh] += lax.dot_general(oh, xl[...], dn, preferred_element_type=F32)
    acc[:, hh:H] += lax.dot_general(oh, xh[...], dn, preferred_element_type=F32)
    cnt[...] += lax.dot_general(
        oh, jnp.ones((t, 1), F32), dn, preferred_element_type=F32
    )

    @pl.when(i == nt - 1)
    def _():
        pooled = acc[...] / jnp.maximum(cnt[...], 1.0)
        o_ref[...] = (
            jnp.dot(pooled, wl[...], preferred_element_type=F32) + bl[...]
        )


@functools.cache
def _pool_kernel(t=2048):
    nt = NP // t
    hh = H // 2
    return pl.pallas_call(
        _pool_body,
        grid=(nt,),
        in_specs=[
            pl.BlockSpec((t, hh), lambda i: (i, 0)),
            pl.BlockSpec((t, hh), lambda i: (NP // t + i, 0)),
            pl.BlockSpec((t, 1), lambda i: (i, 0)),
            pl.BlockSpec((H, C), lambda i: (0, 0)),
            pl.BlockSpec((1, C), lambda i: (0, 0)),
        ],
        out_specs=pl.BlockSpec((G, C), lambda i: (0, 0)),
        out_shape=jax.ShapeDtypeStruct((G, C), F32),
        scratch_shapes=[
            pltpu.VMEM((G, H), F32),
            pltpu.VMEM((G, 1), F32),
        ],
    )


def kernel(x, edge_index, edge_attr, batch,
           We0, be0, W10, b10, W20, b20, g0, bt0,
           We1, be1, W11, b11, W21, b21, g1, bt1,
           We2, be2, W12, b12, W22, b22, g2, bt2,
           Wl, bl):
    src = edge_index[0]
    dst = edge_index[1]
    batch2 = jnp.concatenate(
        [batch, jnp.full((NP - N,), G, jnp.int32)]
    ).reshape(NP, 1)

    e0 = _edge_kernel_full()(edge_attr, We0, be0.reshape(1, D))
    hh = H // 2
    e_splits = []
    for We, be in ((We1, be1), (We2, be2)):
        We_s = jnp.stack([We[:, :hh], We[:, hh:]])
        be_s = jnp.stack([be[:hh], be[hh:]]).reshape(2, 1, hh)
        e_splits.append(_edge_kernel(hh)(edge_attr, We_s, be_s))

    xp = jnp.concatenate([x, jnp.zeros((NP - N, D), F32)], axis=0)
    aggr2 = _sc_aggr_kernel(D, True)(xp, e0, src, dst)
    z, stats = _mlp_kernel0()(
        xp, aggr2, aggr2, W10, b10.reshape(1, H), W20, b20.reshape(1, H)
    )
    x_split = _norm_kernel()(z, stats, g0.reshape(1, H), bt0.reshape(1, H))

    for li, (W1, b1, W2, b2, g, bt) in enumerate((
        (W11, b11, W21, b21, g1, bt1),
        (W12, b12, W22, b22, g2, bt2),
    )):
        aggr_split = _sc_aggr_kernel(hh, False)(x_split, e_splits[li], src, dst)
        z, stats = _mlp_kernel(hh)(
            x_split, x_split, aggr_split, aggr_split,
            W1[:hh], W1[hh:], b1.reshape(1, H), W2, b2.reshape(1, H),
        )
        x_split = _norm_kernel()(z, stats, g.reshape(1, H), bt.reshape(1, H))

    return _pool_kernel()(x_split, x_split, batch2, Wl, bl.reshape(1, C))


# speedup vs baseline: 1.0213x; 1.0140x over previous
import functools

import jax
import jax.numpy as jnp
from jax import lax
from jax.experimental import pallas as pl
from jax.experimental.pallas import tpu as pltpu
from jax.experimental.pallas import tpu_sc as plsc

N = 10000
NP = 10240
E = 320000
D = 128
H = 256
C = 10
G = 64

F32 = jnp.float32


def _edge_body(a_ref, w_ref, b_ref, o_ref):
    o_ref[...] = (
        jnp.dot(a_ref[...], w_ref[0], preferred_element_type=F32) + b_ref[0]
    )


@functools.cache
def _edge_kernel(h, te=2000):
    ne = E // te
    return pl.pallas_call(
        _edge_body,
        grid=(2, ne),
        in_specs=[
            pl.BlockSpec((te, D), lambda c, i: (i, 0)),
            pl.BlockSpec((1, D, h), lambda c, i: (c, 0, 0)),
            pl.BlockSpec((1, 1, h), lambda c, i: (c, 0, 0)),
        ],
        out_specs=pl.BlockSpec((te, h), lambda c, i: (c * ne + i, 0)),
        out_shape=jax.ShapeDtypeStruct((2 * E, h), F32),
    )


def _edge_body_full(a_ref, w_ref, b_ref, o_ref):
    o_ref[...] = (
        jnp.dot(a_ref[...], w_ref[...], preferred_element_type=F32) + b_ref[...]
    )


@functools.cache
def _edge_kernel_full(te=2000):
    ne = E // te
    return pl.pallas_call(
        _edge_body_full,
        grid=(ne,),
        in_specs=[
            pl.BlockSpec((te, D), lambda i: (i, 0)),
            pl.BlockSpec((D, D), lambda i: (0, 0)),
            pl.BlockSpec((1, D), lambda i: (0, 0)),
        ],
        out_specs=pl.BlockSpec((te, D), lambda i: (i, 0)),
        out_shape=jax.ShapeDtypeStruct((E, D), F32),
    )


@functools.cache
def _sc_aggr_kernel(h, edge_split):
    k = 80
    nrows = E // k
    nstr = 32 if edge_split else 16
    nch = nrows // nstr
    rp = NP // 16
    zr = 8
    mesh = plsc.VectorSubcoreMesh(core_axis_name="c", subcore_axis_name="s")

    @functools.partial(
        pl.kernel,
        out_type=jax.ShapeDtypeStruct((2 * NP, h), F32),
        mesh=mesh,
        scratch_types=[
            pltpu.VMEM((4, k), jnp.int32),
            pltpu.VMEM((4, k), jnp.int32),
            pltpu.VMEM((k, h), F32),
            pltpu.VMEM((k, h), F32),
            pltpu.VMEM((k, h), F32),
            pltpu.VMEM((k, h), F32),
            pltpu.VMEM((zr, h), F32),
            pltpu.VMEM_SHARED((NP, h), F32),
            pltpu.SemaphoreType.DMA,
            pltpu.SemaphoreType.DMA,
            pltpu.SemaphoreType.DMA,
            pltpu.SemaphoreType.DMA,
            pltpu.SemaphoreType.DMA,
            pltpu.SemaphoreType.DMA,
            pltpu.SemaphoreType.DMA,
            pltpu.SemaphoreType.DMA,
        ],
    )
    def sc_aggr(x_hbm, e_hbm, src_hbm, dst_hbm, out_hbm,
                srcv, dstv, xg0, xg1, eb0, eb1, zb, aggr,
                sem_is, sem_id, se0, se1, sg0, sg1, ss0, ss1):
        c = lax.axis_index("c")
        s = lax.axis_index("s")
        if edge_split:
            rowbase = c * 16 + s
            ecoff = 0
            coff = 0
        else:
            rowbase = s
            ecoff = c * E
            coff = c * NP

        xgs = (xg0, xg1)
        ebs = (eb0, eb1)
        ses = (se0, se1)
        sgs = (sg0, sg1)
        sss = (ss0, ss1)

        def idx_issue(j):
            off = pl.multiple_of((rowbase + j * nstr) * k, k)
            q = j & 3
            pltpu.async_copy(src_hbm.at[pl.ds(off, k)], srcv.at[q], sem_is)
            pltpu.async_copy(dst_hbm.at[pl.ds(off, k)], dstv.at[q], sem_id)

        def idx_wait():
            pltpu.make_async_copy(src_hbm.at[pl.ds(0, k)], srcv.at[0], sem_is).wait()
            pltpu.make_async_copy(dst_hbm.at[pl.ds(0, k)], dstv.at[0], sem_id).wait()

        def adj(q):
            if not edge_split:
                for jj in range(k // 16):
                    srcv[q, pl.ds(jj * 16, 16)] = (
                        srcv[q, pl.ds(jj * 16, 16)] + coff
                    )

        def gather_issue(j, b):
            pltpu.async_copy(x_hbm.at[srcv.at[j & 3]], xgs[b], sgs[b])

        def e_issue(j, b):
            off = pl.multiple_of(ecoff + (rowbase + j * nstr) * k, k)
            pltpu.async_copy(e_hbm.at[pl.ds(off, k)], ebs[b], ses[b])

        idx_issue(0)
        idx_wait()
        adj(0)
        gather_issue(0, 0)
        e_issue(0, 0)
        idx_issue(1)

        for r in range(zr):
            for jj in range(h // 16):
                zb[r, pl.ds(jj * 16, 16)] = jnp.zeros((16,), F32)

        def zloop(kk, carry):
            pltpu.sync_copy(zb, aggr.at[pl.ds(s * rp + kk * zr, zr)])
            return carry

        lax.fori_loop(0, rp // zr, zloop, 0)
        plsc.subcore_barrier()

        def outer(j0, carry):
            for b in range(2):
                j = j0 * 2 + b
                b1 = 1 - b

                @pl.when(j < nch)
                def _():
                    @pl.when(j + 1 < nch)
                    def _():
                        q1 = (j + 1) & 3
                        idx_wait()
                        adj(q1)
                        gather_issue(j + 1, b1)

                        @pl.when(j + 2 < nch)
                        def _():
                            idx_issue(j + 2)

                        @pl.when(j >= 1)
                        def _():
                            pltpu.make_async_copy(
                                ebs[b1], aggr.at[dstv.at[0]], sss[b1]
                            ).wait()

                        e_issue(j + 1, b1)

                    pltpu.make_async_copy(
                        e_hbm.at[pl.ds(0, k)], ebs[b], ses[b]
                    ).wait()
                    pltpu.make_async_copy(
                        x_hbm.at[srcv.at[0]], xgs[b], sgs[b]
                    ).wait()

                    def rloop(r4, rc):
                        for m in range(4):
                            r = r4 * 4 + m
                            for jj in range(h // 16):
                                v = (
                                    xgs[b][r, pl.ds(jj * 16, 16)]
                                    + ebs[b][r, pl.ds(jj * 16, 16)]
                                )
                                ebs[b][r, pl.ds(jj * 16, 16)] = jnp.maximum(
                                    v, 0.0
                                )
                        return rc

                    lax.fori_loop(0, k // 4, rloop, 0)
                    pltpu.async_copy(
                        ebs[b], aggr.at[dstv.at[j & 3]], sss[b], add=True
                    )
            return carry

        lax.fori_loop(0, (nch + 1) // 2, outer, 0)
        pltpu.make_async_copy(ebs[0], aggr.at[dstv.at[0]], sss[0]).wait()
        pltpu.make_async_copy(ebs[1], aggr.at[dstv.at[0]], sss[1]).wait()
        plsc.subcore_barrier()
        pltpu.sync_copy(
            aggr.at[pl.ds(s * rp, rp)], out_hbm.at[pl.ds(c * NP + s * rp, rp)]
        )

    return sc_aggr


def _mlp_body(xl, xh, al, ah, w1a, w1b, b1, w2, b2, z_ref, st_ref):
    i = pl.program_id(0)
    t = xl.shape[0]
    hin_lo = xl[...] + al[...]
    hin_hi = xh[...] + ah[...]
    h1 = jnp.maximum(
        jnp.dot(hin_lo, w1a[...], preferred_element_type=F32)
        + jnp.dot(hin_hi, w1b[...], preferred_element_type=F32)
        + b1[...],
        0.0,
    )
    z = jnp.dot(h1, w2[...], preferred_element_type=F32) + b2[...]
    rows = lax.broadcasted_iota(jnp.int32, (t, 1), 0) + i * t
    z = jnp.where(rows < N, z, 0.0)
    z_ref[...] = z
    st = jnp.concatenate(
        [jnp.sum(z, 0, keepdims=True), jnp.sum(z * z, 0, keepdims=True)], 0
    )

    @pl.when(i == 0)
    def _():
        st_ref[...] = st

    @pl.when(i > 0)
    def _():
        st_ref[...] += st


@functools.cache
def _mlp_kernel(h, t=2048):
    nt = NP // t
    return pl.pallas_call(
        _mlp_body,
        grid=(nt,),
        in_specs=[
            pl.BlockSpec((t, h), lambda i: (i, 0)),
            pl.BlockSpec((t, h), lambda i: (NP // t + i, 0)),
            pl.BlockSpec((t, h), lambda i: (i, 0)),
            pl.BlockSpec((t, h), lambda i: (NP // t + i, 0)),
            pl.BlockSpec((h, H), lambda i: (0, 0)),
            pl.BlockSpec((h, H), lambda i: (0, 0)),
            pl.BlockSpec((1, H), lambda i: (0, 0)),
            pl.BlockSpec((H, H), lambda i: (0, 0)),
            pl.BlockSpec((1, H), lambda i: (0, 0)),
        ],
        out_specs=[
            pl.BlockSpec((t, H), lambda i: (i, 0)),
            pl.BlockSpec((2, H), lambda i: (0, 0)),
        ],
        out_shape=[
            jax.ShapeDtypeStruct((NP, H), F32),
            jax.ShapeDtypeStruct((2, H), F32),
        ],
    )


def _mlp_body0(xr, a0, a1, w1, b1, w2, b2, z_ref, st_ref):
    i = pl.program_id(0)
    t = xr.shape[0]
    hin = xr[...] + a0[...] + a1[...]
    h1 = jnp.maximum(
        jnp.dot(hin, w1[...], preferred_element_type=F32) + b1[...], 0.0
    )
    z = jnp.dot(h1, w2[...], preferred_element_type=F32) + b2[...]
    rows = lax.broadcasted_iota(jnp.int32, (t, 1), 0) + i * t
    z = jnp.where(rows < N, z, 0.0)
    z_ref[...] = z
    st = jnp.concatenate(
        [jnp.sum(z, 0, keepdims=True), jnp.sum(z * z, 0, keepdims=True)], 0
    )

    @pl.when(i == 0)
    def _():
        st_ref[...] = st

    @pl.when(i > 0)
    def _():
        st_ref[...] += st


@functools.cache
def _mlp_kernel0(t=2048):
    nt = NP // t
    return pl.pallas_call(
        _mlp_body0,
        grid=(nt,),
        in_specs=[
            pl.BlockSpec((t, D), lambda i: (i, 0)),
            pl.BlockSpec((t, D), lambda i: (i, 0)),
            pl.BlockSpec((t, D), lambda i: (NP // t + i, 0)),
            pl.BlockSpec((D, H), lambda i: (0, 0)),
            pl.BlockSpec((1, H), lambda i: (0, 0)),
            pl.BlockSpec((H, H), lambda i: (0, 0)),
            pl.BlockSpec((1, H), lambda i: (0, 0)),
        ],
        out_specs=[
            pl.BlockSpec((t, H), lambda i: (i, 0)),
            pl.BlockSpec((2, H), lambda i: (0, 0)),
        ],
        out_shape=[
            jax.ShapeDtypeStruct((NP, H), F32),
            jax.ShapeDtypeStruct((2, H), F32),
        ],
    )


def _norm_body(z_ref, st_ref, g_ref, bt_ref, o_ref):
    i = pl.program_id(1)
    t = z_ref.shape[0]
    mu = st_ref[0:1, :] * (1.0 / N)
    var = st_ref[1:2, :] * (1.0 / N) - mu * mu
    inv = lax.rsqrt(var + 1e-5)
    o = jnp.maximum((z_ref[...] - mu) * inv * g_ref[...] + bt_ref[...], 0.0)
    rows = lax.broadcasted_iota(jnp.int32, (t, 1), 0) + i * t
    o_ref[...] = jnp.where(rows < N, o, 0.0)


@functools.cache
def _norm_kernel(t=2048):
    nt = NP // t
    hh = H // 2
    return pl.pallas_call(
        _norm_body,
        grid=(2, nt),
        in_specs=[
            pl.BlockSpec((t, hh), lambda c, i: (i, c)),
            pl.BlockSpec((2, hh), lambda c, i: (0, c)),
            pl.BlockSpec((1, hh), lambda c, i: (0, c)),
            pl.BlockSpec((1, hh), lambda c, i: (0, c)),
        ],
        out_specs=pl.BlockSpec((t, hh), lambda c, i: (c * (NP // t) + i, 0)),
        out_shape=jax.ShapeDtypeStruct((2 * NP, hh), F32),
    )


def _pool_body(xl, xh, b_ref, wl, bl, o_ref, acc, cnt):
    i = pl.program_id(0)
    nt = pl.num_programs(0)
    t = b_ref.shape[0]
    hh = H // 2
    oh = (b_ref[...] == lax.broadcasted_iota(jnp.int32, (t, G), 1)).astype(F32)

    @pl.when(i == 0)
    def _():
        acc[...] = jnp.zeros((G, H), F32)
        cnt[...] = jnp.zeros((G, 1), F32)

    dn = (((0,), (0,)), ((), ()))
    acc[:, 0:hh] += lax.dot_general(oh, xl[...], dn, preferred_element_type=F32)
    acc[:, hh:H] += lax.dot_general(oh, xh[...], dn, preferred_element_type=F32)
    cnt[...] += lax.dot_general(
        oh, jnp.ones((t, 1), F32), dn, preferred_element_type=F32
    )

    @pl.when(i == nt - 1)
    def _():
        pooled = acc[...] / jnp.maximum(cnt[...], 1.0)
        o_ref[...] = (
            jnp.dot(pooled, wl[...], preferred_element_type=F32) + bl[...]
        )


@functools.cache
def _pool_kernel(t=2048):
    nt = NP // t
    hh = H // 2
    return pl.pallas_call(
        _pool_body,
        grid=(nt,),
        in_specs=[
            pl.BlockSpec((t, hh), lambda i: (i, 0)),
            pl.BlockSpec((t, hh), lambda i: (NP // t + i, 0)),
            pl.BlockSpec((t, 1), lambda i: (i, 0)),
            pl.BlockSpec((H, C), lambda i: (0, 0)),
            pl.BlockSpec((1, C), lambda i: (0, 0)),
        ],
        out_specs=pl.BlockSpec((G, C), lambda i: (0, 0)),
        out_shape=jax.ShapeDtypeStruct((G, C), F32),
        scratch_shapes=[
            pltpu.VMEM((G, H), F32),
            pltpu.VMEM((G, 1), F32),
        ],
    )


def kernel(x, edge_index, edge_attr, batch,
           We0, be0, W10, b10, W20, b20, g0, bt0,
           We1, be1, W11, b11, W21, b21, g1, bt1,
           We2, be2, W12, b12, W22, b22, g2, bt2,
           Wl, bl):
    src = edge_index[0]
    dst = edge_index[1]
    batch2 = jnp.concatenate(
        [batch, jnp.full((NP - N,), G, jnp.int32)]
    ).reshape(NP, 1)

    ea_bf = edge_attr.astype(jnp.bfloat16)
    e0 = _edge_kernel_full()(
        ea_bf, We0.astype(jnp.bfloat16), be0.reshape(1, D)
    )
    hh = H // 2
    e_splits = []
    for We, be in ((We1, be1), (We2, be2)):
        We_s = jnp.stack([We[:, :hh], We[:, hh:]]).astype(jnp.bfloat16)
        be_s = jnp.stack([be[:hh], be[hh:]]).reshape(2, 1, hh)
        e_splits.append(_edge_kernel(hh)(ea_bf, We_s, be_s))

    xp = jnp.concatenate([x, jnp.zeros((NP - N, D), F32)], axis=0)
    aggr2 = _sc_aggr_kernel(D, True)(xp, e0, src, dst)
    z, stats = _mlp_kernel0()(
        xp, aggr2, aggr2, W10, b10.reshape(1, H), W20, b20.reshape(1, H)
    )
    x_split = _norm_kernel()(z, stats, g0.reshape(1, H), bt0.reshape(1, H))

    for li, (W1, b1, W2, b2, g, bt) in enumerate((
        (W11, b11, W21, b21, g1, bt1),
        (W12, b12, W22, b22, g2, bt2),
    )):
        aggr_split = _sc_aggr_kernel(hh, False)(x_split, e_splits[li], src, dst)
        z, stats = _mlp_kernel(hh)(
            x_split, x_split, aggr_split, aggr_split,
            W1[:hh], W1[hh:], b1.reshape(1, H), W2, b2.reshape(1, H),
        )
        x_split = _norm_kernel()(z, stats, g.reshape(1, H), bt.reshape(1, H))

    return _pool_kernel()(x_split, x_split, batch2, Wl, bl.reshape(1, C))
